# baseline (speedup 1.0000x reference)
import functools

import jax
import jax.numpy as jnp
from jax import lax
from jax.experimental import pallas as pl
from jax.experimental.pallas import tpu as pltpu
from jax.experimental.pallas import tpu_sc as plsc

N_GRAPH, N_NODE, N_FEAT = 256, 128, 9
HIDDEN = 768
W2 = HIDDEN // 2
NUM_ATOMS_P1 = 4609
NUM_IN_DEG = 512
NUM_OUT_DEG = 512

NW = 32
GPW = N_GRAPH // NW
NODES_PW = GPW * N_NODE
K = N_FEAT + 2
C = 8
AROWS = C * N_FEAT
DROWS = C
ROWS_PER_STEP = AROWS + 2 * DROWS
STEPS = NODES_PW // C
STEPS_PER_GRAPH = N_NODE // C
OUT_ROW_STRIDE = (N_NODE + 1) * HIDDEN
LANES = 16
NWRD = W2 // LANES
GUNROLL = 2
HIMASK = jnp.int32(-65536)


def _out_base(wid, s):
    g = wid * GPW + s // STEPS_PER_GRAPH
    n0 = (s % STEPS_PER_GRAPH) * C
    return g * OUT_ROW_STRIDE + (1 + n0) * HIDDEN


def _body(comb_hbm, idx_hbm, token_hbm, out_hbm,
          idx_v, gb0, gb1, ab0, ab1, token_v,
          sg0, sg1, so0, so1):
    wid = lax.axis_index("s") * 2 + lax.axis_index("c")
    g0 = wid * GPW
    gbufs = (gb0, gb1)
    accbs = (ab0, ab1)
    sgs = (sg0, sg1)
    sos = (so0, so1)

    pltpu.sync_copy(idx_hbm.at[pl.ds(wid * NODES_PW * K, NODES_PW * K)], idx_v)

    pltpu.sync_copy(token_hbm, token_v)
    for g in range(GPW):
        pltpu.sync_copy(token_v, out_hbm.at[pl.ds((g0 + g) * OUT_ROW_STRIDE, HIDDEN)])

    def gather(s, p):
        pltpu.async_copy(
            comb_hbm.at[idx_v.at[pl.ds(s * ROWS_PER_STEP, ROWS_PER_STEP)]],
            gbufs[p], sgs[p])

    gather(0, 0)

    def pair(s2, carry):
        for p in (0, 1):
            s = 2 * s2 + p
            q = 1 - p
            @pl.when(s + 1 < STEPS)
            def _():
                gather(s + 1, q)

            pltpu.make_async_copy(comb_hbm.at[pl.ds(0, ROWS_PER_STEP)],
                                  gbufs[p], sgs[p]).wait()

            @pl.when(s2 >= 1)
            def _():
                pltpu.make_async_copy(accbs[p], out_hbm.at[pl.ds(0, C * HIDDEN)],
                                      sos[p]).wait()

            gb = gbufs[p]
            ab = accbs[p]
            for j in range(C):
                def wordg(gg, _, j=j):
                    for u in range(GUNROLL):
                        g = gg * GUNROLL + u
                        gs = pl.ds(g * LANES, LANES)
                        w = gb[j * K, gs]
                        a = plsc.bitcast(w << 16, jnp.float32)
                        b = plsc.bitcast(w & HIMASK, jnp.float32)
                        for t in range(1, K):
                            w = gb[j * K + t, gs]
                            a = a + plsc.bitcast(w << 16, jnp.float32)
                            b = b + plsc.bitcast(w & HIMASK, jnp.float32)
                        ab[pl.ds(j * HIDDEN + g * LANES, LANES)] = a
                        ab[pl.ds(j * HIDDEN + W2 + g * LANES, LANES)] = b
                    return 0

                lax.fori_loop(0, NWRD // GUNROLL, wordg, 0)

            pltpu.async_copy(ab, out_hbm.at[pl.ds(_out_base(wid, s), C * HIDDEN)],
                             sos[p])
        return carry

    lax.fori_loop(0, STEPS // 2, pair, 0)

    for p in (0, 1):
        pltpu.make_async_copy(accbs[p], out_hbm.at[pl.ds(0, C * HIDDEN)],
                              sos[p]).wait()


def _sc_lookup(comb_i32, idx, graph_token):
    mesh = plsc.VectorSubcoreMesh(core_axis_name="c", subcore_axis_name="s")
    fn = functools.partial(
        pl.kernel,
        mesh=mesh,
        compiler_params=pltpu.CompilerParams(needs_layout_passes=False),
        out_type=jax.ShapeDtypeStruct((N_GRAPH * (N_NODE + 1) * HIDDEN,), jnp.float32),
        scratch_types=[
            pltpu.VMEM((NODES_PW * K,), jnp.int32),
            pltpu.VMEM((ROWS_PER_STEP, W2), jnp.int32),
            pltpu.VMEM((ROWS_PER_STEP, W2), jnp.int32),
            pltpu.VMEM((C * HIDDEN,), jnp.float32),
            pltpu.VMEM((C * HIDDEN,), jnp.float32),
            pltpu.VMEM((HIDDEN,), jnp.float32),
            pltpu.SemaphoreType.DMA,
            pltpu.SemaphoreType.DMA,
            pltpu.SemaphoreType.DMA,
            pltpu.SemaphoreType.DMA,
        ],
    )(_body)
    return fn(comb_i32, idx, graph_token.reshape(HIDDEN))


def _pack_body(x_ref, o_ref):
    x = x_ref[...]
    lo = jax.lax.bitcast_convert_type(
        x[:, :W2].astype(jnp.bfloat16), jnp.uint16).astype(jnp.uint32)
    hi = jax.lax.bitcast_convert_type(
        x[:, W2:].astype(jnp.bfloat16), jnp.uint16).astype(jnp.uint32)
    o_ref[...] = jax.lax.bitcast_convert_type(lo | (hi << 16), jnp.int32)


def _pack(table):
    v = table.shape[0]
    blocks = (v + 7) // 8
    return pl.pallas_call(
        _pack_body,
        grid=(blocks,),
        in_specs=[pl.BlockSpec((8, HIDDEN), lambda i: (i, 0))],
        out_specs=pl.BlockSpec((8, W2), lambda i: (i, 0)),
        out_shape=jax.ShapeDtypeStruct((v, W2), jnp.int32),
    )(table)


def kernel(input_nodes, in_degree, out_degree, atom_table, in_deg_table,
           out_deg_table, graph_token):
    comb = jnp.concatenate([atom_table, in_deg_table, out_deg_table], axis=0)
    comb_i32 = _pack(comb)
    idx = jnp.concatenate(
        [
            input_nodes.astype(jnp.int32),
            (in_degree.astype(jnp.int32) + NUM_ATOMS_P1)[..., None],
            (out_degree.astype(jnp.int32) + NUM_ATOMS_P1 + NUM_IN_DEG)[..., None],
        ],
        axis=-1,
    ).reshape(-1)
    flat = _sc_lookup(comb_i32, idx, graph_token)
    return flat.reshape(N_GRAPH, N_NODE + 1, HIDDEN)


# speedup vs baseline: 1.2388x; 1.2388x over previous
import functools

import jax
import jax.numpy as jnp
from jax import lax
from jax.experimental import pallas as pl
from jax.experimental.pallas import tpu as pltpu
from jax.experimental.pallas import tpu_sc as plsc

N_GRAPH, N_NODE, N_FEAT = 256, 128, 9
HIDDEN = 768
W2 = HIDDEN // 2
NUM_IN_DEG = 512
NUM_OUT_DEG = 512
PACK_ROWS = 5632
IN_OFF = 0
OUT_OFF = 512
ATOM_OFF = 1024

NW = 32
GPW = N_GRAPH // NW
NODES_PW = GPW * N_NODE
K = N_FEAT + 2
C = 8
ROWS_PER_STEP = C * K
STEPS = NODES_PW // C
STEPS_PER_GRAPH = N_NODE // C
OUT_ROW_STRIDE = (N_NODE + 1) * HIDDEN
LANES = 16
NGRP = HIDDEN // 32
GUNROLL = 2
PCHUNK = 16
HIMASK = jnp.int32(-65536)
LOMASK = jnp.int32(0xFFFF)


def _out_base(wid, s):
    g = wid * GPW + s // STEPS_PER_GRAPH
    n0 = (s % STEPS_PER_GRAPH) * C
    return g * OUT_ROW_STRIDE + (1 + n0) * HIDDEN


def _body(atab_hbm, itab_hbm, otab_hbm, idx_hbm, token_hbm, out_hbm, pck_hbm,
          idx_v, gb0, gb1, ab0, ab1, fbuf, pbuf, token_v,
          sg0, sg1, so0, so1):
    cid = lax.axis_index("c")
    sid = lax.axis_index("s")
    wid = sid * 2 + cid
    g0 = wid * GPW
    gbufs = (gb0, gb1)
    accbs = (ab0, ab1)
    sgs = (sg0, sg1)
    sos = (so0, so1)
    sc_base = cid * PACK_ROWS

    def pack_arm(src_hbm, base_sid, rows_per_sub, dst_base):
        off = (sid - base_sid) * rows_per_sub

        def ck(c, _):
            src0 = pl.multiple_of(off + c * PCHUNK, 8)
            pltpu.sync_copy(src_hbm.at[pl.ds(src0, PCHUNK)], fbuf)

            def wg(g, _):
                for r in range(PCHUNK):
                    x = plsc.bitcast(fbuf[r, pl.ds(g * 32, LANES)], jnp.int32)
                    y = plsc.bitcast(fbuf[r, pl.ds(g * 32 + LANES, LANES)],
                                     jnp.int32)
                    x2 = x + ((x >> 16) & 1) + 0x7FFF
                    y2 = y + ((y >> 16) & 1) + 0x7FFF
                    w = ((x2 >> 16) & LOMASK) | (y2 & HIMASK)
                    pbuf[r, pl.ds(g * LANES, LANES)] = plsc.bitcast(
                        w, jnp.float32)
                return 0

            lax.fori_loop(0, NGRP, wg, 0)
            dst0 = pl.multiple_of(sc_base + dst_base + off + c * PCHUNK, 8)
            pltpu.sync_copy(pbuf, pck_hbm.at[pl.ds(dst0, PCHUNK)])
            return 0

        lax.fori_loop(0, rows_per_sub // PCHUNK, ck, 0)

    @pl.when(sid < 2)
    def _():
        pack_arm(itab_hbm, 0, NUM_IN_DEG // 2, IN_OFF)

    @pl.when((sid >= 2) & (sid < 4))
    def _():
        pack_arm(otab_hbm, 2, NUM_OUT_DEG // 2, OUT_OFF)

    @pl.when(sid >= 4)
    def _():
        pack_arm(atab_hbm, 4, 4608 // 12, ATOM_OFF)

    pltpu.sync_copy(idx_hbm.at[pl.ds(wid * NODES_PW * K, NODES_PW * K)], idx_v)

    def adj(i, _):
        s16 = pl.ds(i * LANES, LANES)
        idx_v[s16] = idx_v[s16] + sc_base
        return 0

    lax.fori_loop(0, NODES_PW * K // LANES, adj, 0)

    pltpu.sync_copy(token_hbm, token_v)
    for g in range(GPW):
        pltpu.sync_copy(token_v, out_hbm.at[pl.ds((g0 + g) * OUT_ROW_STRIDE, HIDDEN)])

    plsc.subcore_barrier()

    def gather(s, p):
        pltpu.async_copy(
            pck_hbm.at[idx_v.at[pl.ds(s * ROWS_PER_STEP, ROWS_PER_STEP)]],
            gbufs[p], sgs[p])

    gather(0, 0)

    def pair(s2, carry):
        for p in (0, 1):
            s = 2 * s2 + p
            q = 1 - p
            @pl.when(s + 1 < STEPS)
            def _():
                gather(s + 1, q)

            pltpu.make_async_copy(pck_hbm.at[pl.ds(0, ROWS_PER_STEP)],
                                  gbufs[p], sgs[p]).wait()

            @pl.when(s2 >= 1)
            def _():
                pltpu.make_async_copy(accbs[p], out_hbm.at[pl.ds(0, C * HIDDEN)],
                                      sos[p]).wait()

            gb = gbufs[p]
            ab = accbs[p]
            for j in range(C):
                def colg(gg, _, j=j):
                    for u in range(GUNROLL):
                        g = gg * GUNROLL + u
                        gs = pl.ds(g * LANES, LANES)
                        w = plsc.bitcast(gb[j * K, gs], jnp.int32)
                        a = plsc.bitcast(w << 16, jnp.float32)
                        b = plsc.bitcast(w & HIMASK, jnp.float32)
                        for t in range(1, K):
                            w = plsc.bitcast(gb[j * K + t, gs], jnp.int32)
                            a = a + plsc.bitcast(w << 16, jnp.float32)
                            b = b + plsc.bitcast(w & HIMASK, jnp.float32)
                        ab[pl.ds(j * HIDDEN + g * 32, LANES)] = a
                        ab[pl.ds(j * HIDDEN + g * 32 + LANES, LANES)] = b
                    return 0

                lax.fori_loop(0, NGRP // GUNROLL, colg, 0)

            pltpu.async_copy(ab, out_hbm.at[pl.ds(_out_base(wid, s), C * HIDDEN)],
                             sos[p])
        return carry

    lax.fori_loop(0, STEPS // 2, pair, 0)

    for p in (0, 1):
        pltpu.make_async_copy(accbs[p], out_hbm.at[pl.ds(0, C * HIDDEN)],
                              sos[p]).wait()


def _sc_lookup(atab, itab, otab, idx, graph_token):
    mesh = plsc.VectorSubcoreMesh(core_axis_name="c", subcore_axis_name="s")
    fn = functools.partial(
        pl.kernel,
        mesh=mesh,
        compiler_params=pltpu.CompilerParams(needs_layout_passes=False),
        out_type=(
            jax.ShapeDtypeStruct((N_GRAPH * (N_NODE + 1) * HIDDEN,), jnp.float32),
            jax.ShapeDtypeStruct((2 * PACK_ROWS, W2), jnp.float32),
        ),
        scratch_types=[
            pltpu.VMEM((NODES_PW * K,), jnp.int32),
            pltpu.VMEM((ROWS_PER_STEP, W2), jnp.float32),
            pltpu.VMEM((ROWS_PER_STEP, W2), jnp.float32),
            pltpu.VMEM((C * HIDDEN,), jnp.float32),
            pltpu.VMEM((C * HIDDEN,), jnp.float32),
            pltpu.VMEM((PCHUNK, HIDDEN), jnp.float32),
            pltpu.VMEM((PCHUNK, W2), jnp.float32),
            pltpu.VMEM((HIDDEN,), jnp.float32),
            pltpu.SemaphoreType.DMA,
            pltpu.SemaphoreType.DMA,
            pltpu.SemaphoreType.DMA,
            pltpu.SemaphoreType.DMA,
        ],
    )(_body)
    return fn(atab, itab, otab, idx, graph_token.reshape(HIDDEN))


def kernel(input_nodes, in_degree, out_degree, atom_table, in_deg_table,
           out_deg_table, graph_token):
    idx = jnp.concatenate(
        [
            input_nodes.astype(jnp.int32) + ATOM_OFF,
            (in_degree.astype(jnp.int32) + IN_OFF)[..., None],
            (out_degree.astype(jnp.int32) + OUT_OFF)[..., None],
        ],
        axis=-1,
    ).reshape(-1)
    flat, _ = _sc_lookup(atom_table, in_deg_table, out_deg_table, idx,
                         graph_token)
    return flat.reshape(N_GRAPH, N_NODE + 1, HIDDEN)


# speedup vs baseline: 1.3233x; 1.0682x over previous
import functools

import jax
import jax.numpy as jnp
from jax import lax
from jax.experimental import pallas as pl
from jax.experimental.pallas import tpu as pltpu
from jax.experimental.pallas import tpu_sc as plsc

N_GRAPH, N_NODE, N_FEAT = 256, 128, 9
HIDDEN = 768
W2 = HIDDEN // 2
NUM_ATOMS_P1 = 4609
NUM_IN_DEG = 512
NUM_OUT_DEG = 512

NW = 32
GPW = N_GRAPH // NW
NODES_PW = GPW * N_NODE
K = N_FEAT + 2
C = 8
ROWS_PER_STEP = C * K
STEPS = NODES_PW // C
STEPS_PER_GRAPH = N_NODE // C
OUT_ROW_STRIDE = (N_NODE + 1) * HIDDEN
LANES = 16
NGRP = HIDDEN // 32
GUNROLL = 2
HIMASK = jnp.int32(-65536)


def _out_base(wid, s):
    g = wid * GPW + s // STEPS_PER_GRAPH
    n0 = (s % STEPS_PER_GRAPH) * C
    return g * OUT_ROW_STRIDE + (1 + n0) * HIDDEN


def _body(comb_hbm, idx_hbm, token_hbm, out_hbm,
          idx_v, gb0, gb1, ab0, ab1, token_v, sg0, sg1, so0, so1):
    wid = lax.axis_index("s") * 2 + lax.axis_index("c")
    g0 = wid * GPW
    gbufs = (gb0, gb1)
    accbs = (ab0, ab1)
    sgs = (sg0, sg1)
    sos = (so0, so1)

    pltpu.sync_copy(idx_hbm.at[pl.ds(wid * NODES_PW * K, NODES_PW * K)], idx_v)

    pltpu.sync_copy(token_hbm, token_v)
    for g in range(GPW):
        pltpu.sync_copy(token_v, out_hbm.at[pl.ds((g0 + g) * OUT_ROW_STRIDE, HIDDEN)])

    def gather(s, p):
        return pltpu.async_copy(
            comb_hbm.at[idx_v.at[pl.ds(s * ROWS_PER_STEP, ROWS_PER_STEP)]],
            gbufs[p], sgs[p])

    gather(0, 0)

    def pair(s2, carry):
        for p in (0, 1):
            s = 2 * s2 + p
            q = 1 - p
            @pl.when(s + 1 < STEPS)
            def _():
                gather(s + 1, q)

            pltpu.make_async_copy(comb_hbm.at[pl.ds(0, ROWS_PER_STEP)],
                                  gbufs[p], sgs[p]).wait()

            @pl.when(s2 >= 1)
            def _():
                pltpu.make_async_copy(accbs[p], out_hbm.at[pl.ds(0, C * HIDDEN)],
                                      sos[p]).wait()

            gb = gbufs[p]
            ab = accbs[p]
            for j in range(C):
                def colg(gg, _, j=j):
                    for u in range(GUNROLL):
                        g = gg * GUNROLL + u
                        gs = pl.ds(g * LANES, LANES)
                        w = gb[j * K, gs]
                        a = plsc.bitcast(w << 16, jnp.float32)
                        b = plsc.bitcast(w & HIMASK, jnp.float32)
                        for t in range(1, K):
                            w = gb[j * K + t, gs]
                            a = a + plsc.bitcast(w << 16, jnp.float32)
                            b = b + plsc.bitcast(w & HIMASK, jnp.float32)
                        ab[pl.ds(j * HIDDEN + g * 32, LANES)] = a
                        ab[pl.ds(j * HIDDEN + g * 32 + LANES, LANES)] = b
                    return 0

                lax.fori_loop(0, NGRP // GUNROLL, colg, 0)

            pltpu.async_copy(ab, out_hbm.at[pl.ds(_out_base(wid, s), C * HIDDEN)],
                             sos[p])
        return carry

    lax.fori_loop(0, STEPS // 2, pair, 0)

    for p in (0, 1):
        pltpu.make_async_copy(accbs[p], out_hbm.at[pl.ds(0, C * HIDDEN)],
                              sos[p]).wait()


def _sc_lookup(comb_i32, idx, graph_token):
    mesh = plsc.VectorSubcoreMesh(core_axis_name="c", subcore_axis_name="s")
    fn = functools.partial(
        pl.kernel,
        mesh=mesh,
        compiler_params=pltpu.CompilerParams(needs_layout_passes=False),
        out_type=jax.ShapeDtypeStruct((N_GRAPH * (N_NODE + 1) * HIDDEN,), jnp.float32),
        scratch_types=[
            pltpu.VMEM((NODES_PW * K,), jnp.int32),
            pltpu.VMEM((ROWS_PER_STEP, W2), jnp.int32),
            pltpu.VMEM((ROWS_PER_STEP, W2), jnp.int32),
            pltpu.VMEM((C * HIDDEN,), jnp.float32),
            pltpu.VMEM((C * HIDDEN,), jnp.float32),
            pltpu.VMEM((HIDDEN,), jnp.float32),
            pltpu.SemaphoreType.DMA,
            pltpu.SemaphoreType.DMA,
            pltpu.SemaphoreType.DMA,
            pltpu.SemaphoreType.DMA,
        ],
    )(_body)
    return fn(comb_i32, idx, graph_token.reshape(HIDDEN))


def kernel(input_nodes, in_degree, out_degree, atom_table, in_deg_table,
           out_deg_table, graph_token):
    comb = jnp.concatenate([atom_table, in_deg_table, out_deg_table], axis=0)
    v = comb.shape[0]
    u = jax.lax.bitcast_convert_type(comb.astype(jnp.bfloat16), jnp.uint16)
    u = u.astype(jnp.uint32).reshape(v, NGRP, 2, LANES)
    comb_i32 = jax.lax.bitcast_convert_type(
        (u[:, :, 0, :] | (u[:, :, 1, :] << 16)).reshape(v, W2), jnp.int32)
    idx = jnp.concatenate(
        [
            input_nodes.astype(jnp.int32),
            (in_degree.astype(jnp.int32) + NUM_ATOMS_P1)[..., None],
            (out_degree.astype(jnp.int32) + NUM_ATOMS_P1 + NUM_IN_DEG)[..., None],
        ],
        axis=-1,
    ).reshape(-1)
    flat = _sc_lookup(comb_i32, idx, graph_token)
    return flat.reshape(N_GRAPH, N_NODE + 1, HIDDEN)
